# baseline (speedup 1.0000x reference)
import functools

import jax
import jax.numpy as jnp
from jax import lax
from jax.experimental import pallas as pl
from jax.experimental.pallas import tpu as pltpu
from jax.experimental.pallas import tpu_sc as plsc

BATCH = 16384
DIM = 128
CHUNK = 128


@functools.cache
def _build():
    info = plsc.get_sparse_core_info()
    nc, ns = info.num_cores, info.num_subcores
    nw = nc * ns
    b_per_w = BATCH // nw
    ch = b_per_w // CHUNK

    mesh = plsc.VectorSubcoreMesh(core_axis_name="c", subcore_axis_name="s")

    @functools.partial(
        pl.kernel,
        mesh=mesh,
        out_type=jax.ShapeDtypeStruct((BATCH, DIM), jnp.float32),
        scratch_types=[
            pltpu.VMEM((ch, CHUNK), jnp.int32),
            pltpu.VMEM((b_per_w, DIM), jnp.float32),
            pltpu.SemaphoreType.DMA,
            pltpu.SemaphoreType.DMA,
            pltpu.SemaphoreType.DMA,
        ],
    )
    def gather(ids_hbm, tab_hbm, out_hbm, idx_v, rows_v, isem, gsem, wsem):
        wid = lax.axis_index("s") * nc + lax.axis_index("c")
        rbase = wid * ch
        obase = wid * b_per_w

        pltpu.async_copy(ids_hbm.at[pl.ds(rbase, ch)], idx_v, isem).wait()
        gh = [pltpu.async_copy(tab_hbm.at[idx_v.at[j]],
                               rows_v.at[pl.ds(j * CHUNK, CHUNK)], gsem)
              for j in range(ch)]
        for h in gh:
            h.wait()
        pltpu.async_copy(rows_v, out_hbm.at[pl.ds(obase, b_per_w)], wsem).wait()

    return gather


def kernel(user_ids, item_ids, user_table, item_table):
    uids = user_ids.astype(jnp.int32).reshape(BATCH // CHUNK, CHUNK)
    iids = item_ids.astype(jnp.int32).reshape(BATCH // CHUNK, CHUNK)
    g = _build()
    u = g(uids, user_table)
    i = g(iids, item_table)
    u2 = jnp.copy(u)
    return (u, u2, i)


# speedup vs baseline: 1.2556x; 1.2556x over previous
import functools

import jax
import jax.numpy as jnp
from jax import lax
from jax.experimental import pallas as pl
from jax.experimental.pallas import tpu as pltpu
from jax.experimental.pallas import tpu_sc as plsc

BATCH = 16384
DIM = 128
CHUNK = 128


@functools.cache
def _build():
    info = plsc.get_sparse_core_info()
    nc, ns = info.num_cores, info.num_subcores
    nw = nc * ns
    b_per_w = BATCH // nw
    ch = b_per_w // CHUNK

    mesh = plsc.VectorSubcoreMesh(core_axis_name="c", subcore_axis_name="s")

    @functools.partial(
        pl.kernel,
        mesh=mesh,
        out_type=(
            jax.ShapeDtypeStruct((BATCH, DIM), jnp.float32),
            jax.ShapeDtypeStruct((BATCH, DIM), jnp.float32),
            jax.ShapeDtypeStruct((BATCH, DIM), jnp.float32),
        ),
        scratch_types=[
            pltpu.VMEM((ch, CHUNK), jnp.int32),
            pltpu.VMEM((ch, CHUNK), jnp.int32),
            pltpu.VMEM((b_per_w, DIM), jnp.float32),
            pltpu.VMEM((2, CHUNK, DIM), jnp.float32),
            pltpu.SemaphoreType.DMA,
            pltpu.SemaphoreType.DMA,
            pltpu.SemaphoreType.DMA,
            pltpu.SemaphoreType.DMA,
            pltpu.SemaphoreType.DMA,
            pltpu.SemaphoreType.DMA,
            pltpu.SemaphoreType.DMA,
            pltpu.SemaphoreType.DMA,
        ],
    )
    def emb(uids_hbm, iids_hbm, utab_hbm, itab_hbm,
            uout_a_hbm, uout_b_hbm, iout_hbm,
            uidx_v, iidx_v, urows_v, irows_v,
            isem, gu, gi0, gi1, wua, wub, wi0, wi1):
        wid = lax.axis_index("s") * nc + lax.axis_index("c")
        rbase = wid * ch
        obase = wid * b_per_w

        ih_u = pltpu.async_copy(uids_hbm.at[pl.ds(rbase, ch)], uidx_v, isem)
        ih_i = pltpu.async_copy(iids_hbm.at[pl.ds(rbase, ch)], iidx_v, isem)

        gisems = (gi0, gi1)
        wisems = (wi0, wi1)

        ih_u.wait()
        ug = [pltpu.async_copy(utab_hbm.at[uidx_v.at[j]],
                               urows_v.at[pl.ds(j * CHUNK, CHUNK)], gu)
              for j in range(ch)]
        ih_i.wait()
        ig = {j: pltpu.async_copy(itab_hbm.at[iidx_v.at[j]],
                                  irows_v.at[j % 2], gisems[j % 2])
              for j in range(2)}

        iwb = {}

        def item_writeback(j):
            sl = pl.ds(obase + j * CHUNK, CHUNK)
            iwb[j] = pltpu.async_copy(irows_v.at[j % 2], iout_hbm.at[sl],
                                      wisems[j % 2])

        ig[0].wait()
        item_writeback(0)
        ig[1].wait()
        item_writeback(1)

        for h in ug:
            h.wait()
        osl = pl.ds(obase, b_per_w)
        uwa = pltpu.async_copy(urows_v, uout_a_hbm.at[osl], wua)
        uwb = pltpu.async_copy(urows_v, uout_b_hbm.at[osl], wub)

        for j in range(2, ch):
            iwb[j - 2].wait()
            ig[j] = pltpu.async_copy(itab_hbm.at[iidx_v.at[j]],
                                     irows_v.at[j % 2], gisems[j % 2])
        for j in range(2, ch):
            ig[j].wait()
            item_writeback(j)

        uwa.wait()
        uwb.wait()
        for j in range(ch - 2, ch):
            iwb[j].wait()

    return emb


def kernel(user_ids, item_ids, user_table, item_table):
    uids = user_ids.astype(jnp.int32).reshape(BATCH // CHUNK, CHUNK)
    iids = item_ids.astype(jnp.int32).reshape(BATCH // CHUNK, CHUNK)
    return _build()(uids, iids, user_table, item_table)
